# Initial kernel scaffold; baseline (speedup 1.0000x reference)
import jax
import jax.numpy as jnp
from jax.experimental import pallas as pl


def kernel(token_ids, token_table, pos_table):
    raise NotImplementedError("write your pallas kernel here")


# speedup vs baseline: 2.6354x; 2.6354x over previous
import functools

import jax
import jax.numpy as jnp
from jax import lax
from jax.experimental import pallas as pl
from jax.experimental.pallas import tpu as pltpu
from jax.experimental.pallas import tpu_sc as plsc

D_MODEL = 64
SEQ = 200
NUM_CORES = 2
NUM_SUBCORES = 16
NUM_WORKERS = NUM_CORES * NUM_SUBCORES

IDXW = 100
CHUNK = 800
CROWS = CHUNK // IDXW
REPS = CHUNK // SEQ


@functools.partial(jax.jit, static_argnames=("total",))
def _sc_embed(ids2d, token_table, pos_table, *, total):
    tokens_per_worker = total // NUM_WORKERS
    chunks_per_worker = tokens_per_worker // CHUNK
    idx_rows_per_worker = tokens_per_worker // IDXW

    mesh = plsc.VectorSubcoreMesh(
        core_axis_name="c", subcore_axis_name="s",
        num_cores=NUM_CORES, num_subcores=NUM_SUBCORES,
    )

    @functools.partial(
        pl.kernel,
        mesh=mesh,
        compiler_params=pltpu.CompilerParams(use_tc_tiling_on_sc=False),
        out_type=jax.ShapeDtypeStruct((total, D_MODEL), jnp.float32),
        scratch_types=[
            pltpu.VMEM((CROWS, IDXW), jnp.int32),
            pltpu.VMEM((CHUNK, D_MODEL), jnp.float32),
            pltpu.VMEM((SEQ, D_MODEL), jnp.float32),
            pltpu.SemaphoreType.DMA,
        ],
    )
    def body(ids_hbm, table_hbm, pos_hbm, out_hbm, idx_v, rows_v, pos_v, gsem):
        wid = lax.axis_index("s") * NUM_CORES + lax.axis_index("c")
        pltpu.sync_copy(pos_hbm, pos_v)
        base_idx_row = wid * idx_rows_per_worker

        def chunk_body(g, carry):
            crow = base_idx_row + g * CROWS
            pltpu.sync_copy(ids_hbm.at[pl.ds(crow, CROWS)], idx_v)
            copies = [
                pltpu.async_copy(
                    table_hbm.at[idx_v.at[j]],
                    rows_v.at[pl.ds(j * IDXW, IDXW)],
                    gsem,
                )
                for j in range(CROWS)
            ]
            for c in copies:
                c.wait()

            def add_body(r, inner):
                for c in range(D_MODEL // 16):
                    pv = pos_v[r, pl.ds(c * 16, 16)]
                    for rep in range(REPS):
                        row = rep * SEQ + r
                        rows_v[row, pl.ds(c * 16, 16)] = (
                            rows_v[row, pl.ds(c * 16, 16)] + pv
                        )
                return inner

            lax.fori_loop(0, SEQ, add_body, 0, unroll=False)
            pltpu.sync_copy(rows_v, out_hbm.at[pl.ds(crow * IDXW, CHUNK)])
            return carry

        lax.fori_loop(0, chunks_per_worker, chunk_body, 0, unroll=False)

    return body(ids2d, token_table, pos_table)


def kernel(token_ids, token_table, pos_table):
    batch, seq = token_ids.shape
    total = batch * seq
    ids2d = token_ids.reshape(total // IDXW, IDXW).astype(jnp.int32)
    out = _sc_embed(ids2d, token_table, pos_table, total=total)
    return out.reshape(batch, seq, D_MODEL)
